# Initial kernel scaffold; baseline (speedup 1.0000x reference)
import jax
import jax.numpy as jnp
from jax.experimental import pallas as pl


def kernel(x_user, x_item, ei_u2i, ei_i2u, Wsrc_u2i1, bsrc_u2i1, Wdst_u2i1, bdst_u2i1, Wup_u2i1, bup_u2i1, Wsrc_u2i2, bsrc_u2i2, Wdst_u2i2, bdst_u2i2, Wup_u2i2, bup_u2i2, Wsrc_i2u1, bsrc_i2u1, Wdst_i2u1, bdst_i2u1, Wup_i2u1, bup_i2u1, Wsrc_i2u2, bsrc_i2u2, Wdst_i2u2, bdst_i2u2, Wup_i2u2, bup_i2u2, Wpost_user, bpost_user, Wpost_item, bpost_item):
    raise NotImplementedError("write your pallas kernel here")


# speedup vs baseline: 5.6479x; 5.6479x over previous
import functools

import jax
import jax.numpy as jnp
from jax import lax
from jax.experimental import pallas as pl
from jax.experimental.pallas import tpu as pltpu
from jax.experimental.pallas import tpu_sc as plsc

N = 10000
E = 320000
D = 128

NC = 2
NS = 16
NT = NC * NS
EPT = E // NT
CHUNK = 80
NCH = EPT // CHUNK
NPAD = 10240
STRIPE = NPAD // NS
ZROWS = 128

_F32 = jnp.float32


def _wid():
    return lax.axis_index("s") * NC + lax.axis_index("c")


DH = D // 2
SEPT = E // NS
SNCH = SEPT // CHUNK


def _make_segsum():
    mesh = plsc.VectorSubcoreMesh(core_axis_name="c", subcore_axis_name="s", num_cores=NC, num_subcores=NS)

    @functools.partial(
        pl.kernel,
        mesh=mesh,
        compiler_params=pltpu.CompilerParams(use_tc_tiling_on_sc=False),
        out_type=[
            jax.ShapeDtypeStruct((NC, NPAD, DH), _F32),
            jax.ShapeDtypeStruct((NC, NPAD, 16), _F32),
        ],
        scratch_types=[
            pltpu.VMEM((SNCH, CHUNK), jnp.int32),
            pltpu.VMEM((SNCH, CHUNK), jnp.int32),
            pltpu.VMEM((CHUNK, DH), _F32),
            pltpu.VMEM((CHUNK, DH), _F32),
            pltpu.VMEM((CHUNK, 16), _F32),
            pltpu.VMEM((ZROWS, DH), _F32),
            pltpu.VMEM((ZROWS, 16), _F32),
            pltpu.VMEM_SHARED((NPAD, DH), _F32),
            pltpu.VMEM_SHARED((NPAD, 16), _F32),
            pltpu.SemaphoreType.DMA,
            pltpu.SemaphoreType.DMA,
        ],
    )
    def seg(x_hbm, si_hbm, di_hbm, out_hbm, cnt_hbm,
            si_v, di_v, buf0, buf1, ones_v, zb, zb16, acc_sh, cnt_sh,
            sem0, sem1):
        c = lax.axis_index("c")
        s = lax.axis_index("s")
        z16 = jnp.zeros((16,), _F32)
        o16 = jnp.ones((16,), _F32)

        @pl.loop(0, ZROWS)
        def _(i):
            for k in range(DH // 16):
                zb[i, pl.ds(16 * k, 16)] = z16
            zb16[i, :] = z16

        @pl.loop(0, CHUNK)
        def _(i):
            ones_v[i, :] = o16

        for q in range(STRIPE // ZROWS):
            base = s * STRIPE + q * ZROWS
            pltpu.sync_copy(zb, acc_sh.at[pl.ds(base, ZROWS)])
            pltpu.sync_copy(zb16, cnt_sh.at[pl.ds(base, ZROWS)])
        plsc.subcore_barrier()

        pltpu.sync_copy(si_hbm.at[s], si_v)
        pltpu.sync_copy(di_hbm.at[s], di_v)

        bufs = (buf0, buf1)
        sems = (sem0, sem1)
        xh = x_hbm.at[c]

        for b in range(2):
            pltpu.make_async_copy(xh.at[si_v.at[b]], bufs[b], sems[b]).start()

        @pl.loop(0, SNCH, step=2)
        def _(jj):
            for b in range(2):
                j = jj + b
                pltpu.make_async_copy(
                    xh.at[si_v.at[j]], bufs[b], sems[b]).wait()
                pltpu.sync_copy(bufs[b], acc_sh.at[di_v.at[j]], add=True)

                @pl.when(c == 0)
                def _():
                    pltpu.sync_copy(ones_v, cnt_sh.at[di_v.at[j]], add=True)

                @pl.when(j + 2 < SNCH)
                def _():
                    pltpu.make_async_copy(
                        xh.at[si_v.at[j + 2]], bufs[b], sems[b]).start()

        plsc.subcore_barrier()
        rbase = s * STRIPE
        pltpu.sync_copy(acc_sh.at[pl.ds(rbase, STRIPE)],
                        out_hbm.at[c, pl.ds(rbase, STRIPE)])
        pltpu.sync_copy(cnt_sh.at[pl.ds(rbase, STRIPE)],
                        cnt_hbm.at[c, pl.ds(rbase, STRIPE)])

    return seg


def _make_edgedot():
    mesh = plsc.VectorSubcoreMesh(core_axis_name="c", subcore_axis_name="s", num_cores=NC, num_subcores=NS)

    @functools.partial(
        pl.kernel,
        mesh=mesh,
        out_type=[
            jax.ShapeDtypeStruct((E, 16), _F32),
            jax.ShapeDtypeStruct((E, 16), _F32),
        ],
        scratch_types=[
            pltpu.VMEM((NCH, CHUNK), jnp.int32),
            pltpu.VMEM((NCH, CHUNK), jnp.int32),
            pltpu.VMEM((CHUNK, D), _F32),
            pltpu.VMEM((CHUNK, D), _F32),
            pltpu.VMEM((CHUNK, D), _F32),
            pltpu.VMEM((CHUNK, D), _F32),
            pltpu.VMEM((CHUNK, 16), _F32),
            pltpu.VMEM((CHUNK, 16), _F32),
            pltpu.SemaphoreType.DMA,
            pltpu.SemaphoreType.DMA,
            pltpu.SemaphoreType.DMA,
            pltpu.SemaphoreType.DMA,
        ],
    )
    def edot(u_hbm, it_hbm, ia0_hbm, ib0_hbm, ia1_hbm, ib1_hbm,
             o0_hbm, o1_hbm,
             ia_v, ib_v, ra0, ra1, rb0, rb1, po0, po1,
             sa0, sa1, sb0, sb1):
        wid = _wid()
        base = wid * EPT
        ras = (ra0, ra1)
        rbs = (rb0, rb1)
        pos = (po0, po1)
        sas = (sa0, sa1)
        sbs = (sb0, sb1)

        for tab_a, tab_b, ia_h, ib_h, o_h in (
                (u_hbm, it_hbm, ia0_hbm, ib0_hbm, o0_hbm),
                (it_hbm, u_hbm, ia1_hbm, ib1_hbm, o1_hbm)):
            pltpu.sync_copy(ia_h.at[wid], ia_v)
            pltpu.sync_copy(ib_h.at[wid], ib_v)
            for b in range(2):
                pltpu.make_async_copy(
                    tab_a.at[ia_v.at[b]], ras[b], sas[b]).start()
                pltpu.make_async_copy(
                    tab_b.at[ib_v.at[b]], rbs[b], sbs[b]).start()

            def _chunk(j, b):
                pltpu.make_async_copy(
                    tab_a.at[ia_v.at[j]], ras[b], sas[b]).wait()
                pltpu.make_async_copy(
                    tab_b.at[ib_v.at[j]], rbs[b], sbs[b]).wait()
                ra, rb, po = ras[b], rbs[b], pos[b]

                @pl.loop(0, CHUNK)
                def _(e):
                    acc = ra[e, pl.ds(0, 16)] * rb[e, pl.ds(0, 16)]
                    for k in range(1, D // 16):
                        acc = acc + (ra[e, pl.ds(16 * k, 16)]
                                     * rb[e, pl.ds(16 * k, 16)])
                    po[e, :] = acc

                pltpu.sync_copy(po, o_h.at[pl.ds(base + j * CHUNK, CHUNK)])

            @pl.loop(0, NCH - 1, step=2)
            def _(jj):
                for b in range(2):
                    j = jj + b
                    _chunk(j, b)

                    @pl.when(j + 2 < NCH)
                    def _():
                        pltpu.make_async_copy(
                            tab_a.at[ia_v.at[j + 2]], ras[b], sas[b]).start()
                        pltpu.make_async_copy(
                            tab_b.at[ib_v.at[j + 2]], rbs[b], sbs[b]).start()

            _chunk(NCH - 1, 0)

    return edot


_BN = 1000
_GRID = N // _BN
_HI = jax.lax.Precision.HIGHEST


def _dense_a_body(parts, cnts, xdst, wsrc, bsrc, wdst, bdst, wup, bup,
                  z_ref, st_ref):
    ssum = jnp.concatenate([parts[0], parts[1]], axis=1)
    cnt = cnts[0, :, 0:1]
    aggr = ssum / jnp.maximum(cnt, 1.0)
    msgs = jnp.dot(aggr, wsrc[...].T, precision=_HI) + bsrc[...]
    owns = jnp.dot(xdst[...], wdst[...].T, precision=_HI) + bdst[...]
    wu = wup[...]
    z = (jnp.dot(owns, wu[:, :D].T, precision=_HI)
         + jnp.dot(msgs, wu[:, D:].T, precision=_HI) + bup[...])
    z_ref[...] = z

    @pl.when(pl.program_id(0) == 0)
    def _():
        st_ref[...] = jnp.zeros_like(st_ref)

    st_ref[...] = st_ref[...] + jnp.concatenate(
        [jnp.sum(z, axis=0, keepdims=True),
         jnp.sum(z * z, axis=0, keepdims=True)], axis=0)


_DENSE_A = pl.pallas_call(
    _dense_a_body,
    grid=(_GRID,),
    in_specs=[
        pl.BlockSpec((2, _BN, D // 2), lambda i: (0, i, 0)),
        pl.BlockSpec((1, _BN, 16), lambda i: (0, i, 0)),
        pl.BlockSpec((_BN, D), lambda i: (i, 0)),
        pl.BlockSpec((D, D), lambda i: (0, 0)),
        pl.BlockSpec((1, D), lambda i: (0, 0)),
        pl.BlockSpec((D, D), lambda i: (0, 0)),
        pl.BlockSpec((1, D), lambda i: (0, 0)),
        pl.BlockSpec((D, 2 * D), lambda i: (0, 0)),
        pl.BlockSpec((1, D), lambda i: (0, 0)),
    ],
    out_specs=[
        pl.BlockSpec((_BN, D), lambda i: (i, 0)),
        pl.BlockSpec((2, D), lambda i: (0, 0)),
    ],
    out_shape=[
        jax.ShapeDtypeStruct((N, D), _F32),
        jax.ShapeDtypeStruct((2, D), _F32),
    ],
)


def _dense_b1_body(z_ref, st_ref, out_ref):
    m = st_ref[0:1, :] / N
    v = st_ref[1:2, :] / N - m * m
    hn = (z_ref[...] - m) / jnp.sqrt(v + 1.0)
    out_ref[...] = jnp.where(hn >= 0, hn, 0.01 * hn)


_DENSE_B1 = pl.pallas_call(
    _dense_b1_body,
    grid=(_GRID,),
    in_specs=[
        pl.BlockSpec((_BN, D), lambda i: (i, 0)),
        pl.BlockSpec((2, D), lambda i: (0, 0)),
    ],
    out_specs=pl.BlockSpec((_BN, D), lambda i: (i, 0)),
    out_shape=jax.ShapeDtypeStruct((N, D), _F32),
)


def _dense_b2_body(z_ref, st_ref, wp_ref, bp_ref, out_ref):
    m = st_ref[0:1, :] / N
    v = st_ref[1:2, :] / N - m * m
    hn = (z_ref[...] - m) / jnp.sqrt(v + 1.0)
    h = jnp.where(hn >= 0, hn, 0.01 * hn)
    out_ref[...] = jnp.dot(h, wp_ref[...].T, precision=_HI) + bp_ref[...]


_DENSE_B2 = pl.pallas_call(
    _dense_b2_body,
    grid=(_GRID,),
    in_specs=[
        pl.BlockSpec((_BN, D), lambda i: (i, 0)),
        pl.BlockSpec((2, D), lambda i: (0, 0)),
        pl.BlockSpec((D, D), lambda i: (0, 0)),
        pl.BlockSpec((1, D), lambda i: (0, 0)),
    ],
    out_specs=pl.BlockSpec((_BN, D), lambda i: (i, 0)),
    out_shape=jax.ShapeDtypeStruct((N, D), _F32),
)

_EROWS = E // 8
_BP = _EROWS // 8


def _pred_body(pa_ref, pb_ref, oa_ref, ob_ref):
    r = lax.broadcasted_iota(jnp.int32, (D, 8), 0) // 16
    cidx = lax.broadcasted_iota(jnp.int32, (D, 8), 1)
    sel = (r == cidx).astype(_F32)
    for pref, oref in ((pa_ref, oa_ref), (pb_ref, ob_ref)):
        t = jnp.dot(pref[...], sel, precision=_HI)
        oref[...] = jax.nn.sigmoid(t)


_PRED = pl.pallas_call(
    _pred_body,
    grid=(8,),
    in_specs=[
        pl.BlockSpec((_BP, D), lambda i: (i, 0)),
        pl.BlockSpec((_BP, D), lambda i: (i, 0)),
    ],
    out_specs=[
        pl.BlockSpec((_BP, 8), lambda i: (i, 0)),
        pl.BlockSpec((_BP, 8), lambda i: (i, 0)),
    ],
    out_shape=[
        jax.ShapeDtypeStruct((_EROWS, 8), _F32),
        jax.ShapeDtypeStruct((_EROWS, 8), _F32),
    ],
)

_SEGSUM = functools.cache(_make_segsum)
_EDGEDOT = functools.cache(_make_edgedot)


def kernel(x_user, x_item, ei_u2i, ei_i2u,
           Wsrc_u2i1, bsrc_u2i1, Wdst_u2i1, bdst_u2i1, Wup_u2i1, bup_u2i1,
           Wsrc_u2i2, bsrc_u2i2, Wdst_u2i2, bdst_u2i2, Wup_u2i2, bup_u2i2,
           Wsrc_i2u1, bsrc_i2u1, Wdst_i2u1, bdst_i2u1, Wup_i2u1, bup_i2u1,
           Wsrc_i2u2, bsrc_i2u2, Wdst_i2u2, bdst_i2u2, Wup_i2u2, bup_i2u2,
           Wpost_user, bpost_user, Wpost_item, bpost_item):
    r2 = lambda b: b.reshape(1, D)
    halves = lambda x: jnp.stack([x[:, :DH], x[:, DH:]])
    e0u = ei_u2i[0].reshape(NS, SNCH, CHUNK)
    e1u = ei_u2i[1].reshape(NS, SNCH, CHUNK)
    e0i = ei_i2u[0].reshape(NS, SNCH, CHUNK)
    e1i = ei_i2u[1].reshape(NS, SNCH, CHUNK)
    f0u = ei_u2i[0].reshape(NT, NCH, CHUNK)
    f1u = ei_u2i[1].reshape(NT, NCH, CHUNK)
    f0i = ei_i2u[0].reshape(NT, NCH, CHUNK)
    f1i = ei_i2u[1].reshape(NT, NCH, CHUNK)

    seg = _SEGSUM()
    edot = _EDGEDOT()
    ps_it, pc_u2i = seg(halves(x_user), e0u, e1u)
    ps_us, pc_i2u = seg(halves(x_item), e0i, e1i)
    z_it, st_it = _DENSE_A(ps_it, pc_u2i, x_item, Wsrc_u2i1, r2(bsrc_u2i1),
                           Wdst_u2i1, r2(bdst_u2i1), Wup_u2i1, r2(bup_u2i1))
    z_us, st_us = _DENSE_A(ps_us, pc_i2u, x_user, Wsrc_i2u1, r2(bsrc_i2u1),
                           Wdst_i2u1, r2(bdst_i2u1), Wup_i2u1, r2(bup_i2u1))
    h_it = _DENSE_B1(z_it, st_it)
    h_us = _DENSE_B1(z_us, st_us)

    ps_it2, pc2 = seg(halves(h_us), e0u, e1u)
    ps_us2, pc3 = seg(halves(h_it), e0i, e1i)
    z_it2, st_it2 = _DENSE_A(ps_it2, pc2, h_it, Wsrc_u2i2, r2(bsrc_u2i2),
                             Wdst_u2i2, r2(bdst_u2i2), Wup_u2i2, r2(bup_u2i2))
    z_us2, st_us2 = _DENSE_A(ps_us2, pc3, h_us, Wsrc_i2u2, r2(bsrc_i2u2),
                             Wdst_i2u2, r2(bdst_i2u2), Wup_i2u2, r2(bup_i2u2))
    it = _DENSE_B2(z_it2, st_it2, Wpost_item, r2(bpost_item))
    u = _DENSE_B2(z_us2, st_us2, Wpost_user, r2(bpost_user))

    p0, p1 = edot(u, it, f0u, f1u, f0i, f1i)
    pr0, pr1 = _PRED(p0.reshape(_EROWS, D), p1.reshape(_EROWS, D))
    return pr0.reshape(E), pr1.reshape(E)


# speedup vs baseline: 6.5239x; 1.1551x over previous
import functools

import jax
import jax.numpy as jnp
from jax import lax
from jax.experimental import pallas as pl
from jax.experimental.pallas import tpu as pltpu
from jax.experimental.pallas import tpu_sc as plsc

N = 10000
E = 320000
D = 128

NC = 2
NS = 16
NT = NC * NS
EPT = E // NT
CHUNK = 80
NCH = EPT // CHUNK
NPAD = 10240
STRIPE = NPAD // NS
ZROWS = 128

_F32 = jnp.float32


def _wid():
    return lax.axis_index("s") * NC + lax.axis_index("c")


DH = D // 2
SEPT = E // NS
SNCH = SEPT // CHUNK


def _make_segsum():
    mesh = plsc.VectorSubcoreMesh(core_axis_name="c", subcore_axis_name="s", num_cores=NC, num_subcores=NS)

    @functools.partial(
        pl.kernel,
        mesh=mesh,
        compiler_params=pltpu.CompilerParams(use_tc_tiling_on_sc=False),
        out_type=[
            jax.ShapeDtypeStruct((NC, NPAD, DH), _F32),
            jax.ShapeDtypeStruct((NC, NPAD, 16), _F32),
        ],
        scratch_types=[
            pltpu.VMEM((SNCH, CHUNK), jnp.int32),
            pltpu.VMEM((SNCH, CHUNK), jnp.int32),
            [pltpu.VMEM((CHUNK, DH), _F32) for _ in range(5)],
            pltpu.VMEM((CHUNK, 16), _F32),
            pltpu.VMEM((ZROWS, DH), _F32),
            pltpu.VMEM((ZROWS, 16), _F32),
            pltpu.VMEM_SHARED((NPAD, DH), _F32),
            pltpu.VMEM_SHARED((NPAD, 16), _F32),
            [pltpu.SemaphoreType.DMA for _ in range(5)],
            [pltpu.SemaphoreType.DMA for _ in range(5)],
        ],
    )
    def seg(x_hbm, si_hbm, di_hbm, out_hbm, cnt_hbm,
            si_v, di_v, bufs, ones_v, zb, zb16, acc_sh, cnt_sh,
            gsems, ssems):
        NB = 5
        c = lax.axis_index("c")
        s = lax.axis_index("s")
        z16 = jnp.zeros((16,), _F32)
        o16 = jnp.ones((16,), _F32)

        @pl.loop(0, ZROWS)
        def _(i):
            for k in range(DH // 16):
                zb[i, pl.ds(16 * k, 16)] = z16
            zb16[i, :] = z16

        @pl.loop(0, CHUNK)
        def _(i):
            ones_v[i, :] = o16

        for q in range(STRIPE // ZROWS):
            base = s * STRIPE + q * ZROWS
            pltpu.sync_copy(zb, acc_sh.at[pl.ds(base, ZROWS)])
            pltpu.sync_copy(zb16, cnt_sh.at[pl.ds(base, ZROWS)])
        plsc.subcore_barrier()

        pltpu.sync_copy(si_hbm.at[s], si_v)
        pltpu.sync_copy(di_hbm.at[s], di_v)

        xh = x_hbm.at[c]

        for b in range(NB):
            pltpu.make_async_copy(xh.at[si_v.at[b]], bufs[b], gsems[b]).start()

        @pl.loop(0, SNCH, step=NB)
        def _(jj):
            for b in range(NB):
                j = jj + b
                pltpu.make_async_copy(
                    xh.at[si_v.at[j]], bufs[b], gsems[b]).wait()
                pltpu.async_copy(bufs[b], acc_sh.at[di_v.at[j]], ssems[b],
                                 add=True)

                @pl.when(c == 0)
                def _():
                    pltpu.async_copy(ones_v, cnt_sh.at[di_v.at[j]], ssems[b],
                                     add=True)

            for b in range(NB):
                j2 = jj + NB + b

                @pl.when(j2 < SNCH)
                def _():
                    pltpu.make_async_copy(
                        bufs[b], acc_sh.at[di_v.at[b]], ssems[b]).wait()

                    @pl.when(c == 0)
                    def _():
                        pltpu.make_async_copy(
                            ones_v, cnt_sh.at[di_v.at[b]], ssems[b]).wait()

                    pltpu.make_async_copy(
                        xh.at[si_v.at[j2]], bufs[b], gsems[b]).start()

        for b in range(NB):
            pltpu.make_async_copy(bufs[b], acc_sh.at[di_v.at[b]],
                                  ssems[b]).wait()

            @pl.when(c == 0)
            def _():
                pltpu.make_async_copy(ones_v, cnt_sh.at[di_v.at[b]],
                                      ssems[b]).wait()

        plsc.subcore_barrier()
        rbase = s * STRIPE
        pltpu.sync_copy(acc_sh.at[pl.ds(rbase, STRIPE)],
                        out_hbm.at[c, pl.ds(rbase, STRIPE)])
        pltpu.sync_copy(cnt_sh.at[pl.ds(rbase, STRIPE)],
                        cnt_hbm.at[c, pl.ds(rbase, STRIPE)])

    return seg


def _make_edgedot():
    mesh = plsc.VectorSubcoreMesh(core_axis_name="c", subcore_axis_name="s", num_cores=NC, num_subcores=NS)

    @functools.partial(
        pl.kernel,
        mesh=mesh,
        out_type=[
            jax.ShapeDtypeStruct((E, 16), _F32),
            jax.ShapeDtypeStruct((E, 16), _F32),
        ],
        scratch_types=[
            pltpu.VMEM((NCH, CHUNK), jnp.int32),
            pltpu.VMEM((NCH, CHUNK), jnp.int32),
            pltpu.VMEM((CHUNK, D), _F32),
            pltpu.VMEM((CHUNK, D), _F32),
            pltpu.VMEM((CHUNK, D), _F32),
            pltpu.VMEM((CHUNK, D), _F32),
            pltpu.VMEM((CHUNK, 16), _F32),
            pltpu.VMEM((CHUNK, 16), _F32),
            pltpu.SemaphoreType.DMA,
            pltpu.SemaphoreType.DMA,
            pltpu.SemaphoreType.DMA,
            pltpu.SemaphoreType.DMA,
        ],
    )
    def edot(u_hbm, it_hbm, ia0_hbm, ib0_hbm, ia1_hbm, ib1_hbm,
             o0_hbm, o1_hbm,
             ia_v, ib_v, ra0, ra1, rb0, rb1, po0, po1,
             sa0, sa1, sb0, sb1):
        wid = _wid()
        base = wid * EPT
        ras = (ra0, ra1)
        rbs = (rb0, rb1)
        pos = (po0, po1)
        sas = (sa0, sa1)
        sbs = (sb0, sb1)

        for tab_a, tab_b, ia_h, ib_h, o_h in (
                (u_hbm, it_hbm, ia0_hbm, ib0_hbm, o0_hbm),
                (it_hbm, u_hbm, ia1_hbm, ib1_hbm, o1_hbm)):
            pltpu.sync_copy(ia_h.at[wid], ia_v)
            pltpu.sync_copy(ib_h.at[wid], ib_v)
            for b in range(2):
                pltpu.make_async_copy(
                    tab_a.at[ia_v.at[b]], ras[b], sas[b]).start()
                pltpu.make_async_copy(
                    tab_b.at[ib_v.at[b]], rbs[b], sbs[b]).start()

            def _chunk(j, b):
                pltpu.make_async_copy(
                    tab_a.at[ia_v.at[j]], ras[b], sas[b]).wait()
                pltpu.make_async_copy(
                    tab_b.at[ib_v.at[j]], rbs[b], sbs[b]).wait()
                ra, rb, po = ras[b], rbs[b], pos[b]

                @pl.loop(0, CHUNK)
                def _(e):
                    acc = ra[e, pl.ds(0, 16)] * rb[e, pl.ds(0, 16)]
                    for k in range(1, D // 16):
                        acc = acc + (ra[e, pl.ds(16 * k, 16)]
                                     * rb[e, pl.ds(16 * k, 16)])
                    po[e, :] = acc

                pltpu.sync_copy(po, o_h.at[pl.ds(base + j * CHUNK, CHUNK)])

            @pl.loop(0, NCH - 1, step=2)
            def _(jj):
                for b in range(2):
                    j = jj + b
                    _chunk(j, b)

                    @pl.when(j + 2 < NCH)
                    def _():
                        pltpu.make_async_copy(
                            tab_a.at[ia_v.at[j + 2]], ras[b], sas[b]).start()
                        pltpu.make_async_copy(
                            tab_b.at[ib_v.at[j + 2]], rbs[b], sbs[b]).start()

            _chunk(NCH - 1, 0)

    return edot


_BN = 1000
_GRID = N // _BN
_HI = jax.lax.Precision.HIGHEST


def _dense_a_body(parts, cnts, xdst, wsrc, bsrc, wdst, bdst, wup, bup,
                  z_ref, st_ref):
    ssum = jnp.concatenate([parts[0], parts[1]], axis=1)
    cnt = cnts[0, :, 0:1]
    aggr = ssum / jnp.maximum(cnt, 1.0)
    msgs = jnp.dot(aggr, wsrc[...].T, precision=_HI) + bsrc[...]
    owns = jnp.dot(xdst[...], wdst[...].T, precision=_HI) + bdst[...]
    wu = wup[...]
    z = (jnp.dot(owns, wu[:, :D].T, precision=_HI)
         + jnp.dot(msgs, wu[:, D:].T, precision=_HI) + bup[...])
    z_ref[...] = z

    @pl.when(pl.program_id(0) == 0)
    def _():
        st_ref[...] = jnp.zeros_like(st_ref)

    st_ref[...] = st_ref[...] + jnp.concatenate(
        [jnp.sum(z, axis=0, keepdims=True),
         jnp.sum(z * z, axis=0, keepdims=True)], axis=0)


_DENSE_A = pl.pallas_call(
    _dense_a_body,
    grid=(_GRID,),
    in_specs=[
        pl.BlockSpec((2, _BN, D // 2), lambda i: (0, i, 0)),
        pl.BlockSpec((1, _BN, 16), lambda i: (0, i, 0)),
        pl.BlockSpec((_BN, D), lambda i: (i, 0)),
        pl.BlockSpec((D, D), lambda i: (0, 0)),
        pl.BlockSpec((1, D), lambda i: (0, 0)),
        pl.BlockSpec((D, D), lambda i: (0, 0)),
        pl.BlockSpec((1, D), lambda i: (0, 0)),
        pl.BlockSpec((D, 2 * D), lambda i: (0, 0)),
        pl.BlockSpec((1, D), lambda i: (0, 0)),
    ],
    out_specs=[
        pl.BlockSpec((_BN, D), lambda i: (i, 0)),
        pl.BlockSpec((2, D), lambda i: (0, 0)),
    ],
    out_shape=[
        jax.ShapeDtypeStruct((N, D), _F32),
        jax.ShapeDtypeStruct((2, D), _F32),
    ],
)


def _dense_b1_body(z_ref, st_ref, out_ref):
    m = st_ref[0:1, :] / N
    v = st_ref[1:2, :] / N - m * m
    hn = (z_ref[...] - m) / jnp.sqrt(v + 1.0)
    out_ref[...] = jnp.where(hn >= 0, hn, 0.01 * hn)


_DENSE_B1 = pl.pallas_call(
    _dense_b1_body,
    grid=(_GRID,),
    in_specs=[
        pl.BlockSpec((_BN, D), lambda i: (i, 0)),
        pl.BlockSpec((2, D), lambda i: (0, 0)),
    ],
    out_specs=pl.BlockSpec((_BN, D), lambda i: (i, 0)),
    out_shape=jax.ShapeDtypeStruct((N, D), _F32),
)


def _dense_b2_body(z_ref, st_ref, wp_ref, bp_ref, out_ref):
    m = st_ref[0:1, :] / N
    v = st_ref[1:2, :] / N - m * m
    hn = (z_ref[...] - m) / jnp.sqrt(v + 1.0)
    h = jnp.where(hn >= 0, hn, 0.01 * hn)
    out_ref[...] = jnp.dot(h, wp_ref[...].T, precision=_HI) + bp_ref[...]


_DENSE_B2 = pl.pallas_call(
    _dense_b2_body,
    grid=(_GRID,),
    in_specs=[
        pl.BlockSpec((_BN, D), lambda i: (i, 0)),
        pl.BlockSpec((2, D), lambda i: (0, 0)),
        pl.BlockSpec((D, D), lambda i: (0, 0)),
        pl.BlockSpec((1, D), lambda i: (0, 0)),
    ],
    out_specs=pl.BlockSpec((_BN, D), lambda i: (i, 0)),
    out_shape=jax.ShapeDtypeStruct((N, D), _F32),
)

_EROWS = E // 8
_BP = _EROWS // 8


def _pred_body(pa_ref, pb_ref, oa_ref, ob_ref):
    r = lax.broadcasted_iota(jnp.int32, (D, 8), 0) // 16
    cidx = lax.broadcasted_iota(jnp.int32, (D, 8), 1)
    sel = (r == cidx).astype(_F32)
    for pref, oref in ((pa_ref, oa_ref), (pb_ref, ob_ref)):
        t = jnp.dot(pref[...], sel, precision=_HI)
        oref[...] = jax.nn.sigmoid(t)


_PRED = pl.pallas_call(
    _pred_body,
    grid=(8,),
    in_specs=[
        pl.BlockSpec((_BP, D), lambda i: (i, 0)),
        pl.BlockSpec((_BP, D), lambda i: (i, 0)),
    ],
    out_specs=[
        pl.BlockSpec((_BP, 8), lambda i: (i, 0)),
        pl.BlockSpec((_BP, 8), lambda i: (i, 0)),
    ],
    out_shape=[
        jax.ShapeDtypeStruct((_EROWS, 8), _F32),
        jax.ShapeDtypeStruct((_EROWS, 8), _F32),
    ],
)

_SEGSUM = functools.cache(_make_segsum)
_EDGEDOT = functools.cache(_make_edgedot)


def kernel(x_user, x_item, ei_u2i, ei_i2u,
           Wsrc_u2i1, bsrc_u2i1, Wdst_u2i1, bdst_u2i1, Wup_u2i1, bup_u2i1,
           Wsrc_u2i2, bsrc_u2i2, Wdst_u2i2, bdst_u2i2, Wup_u2i2, bup_u2i2,
           Wsrc_i2u1, bsrc_i2u1, Wdst_i2u1, bdst_i2u1, Wup_i2u1, bup_i2u1,
           Wsrc_i2u2, bsrc_i2u2, Wdst_i2u2, bdst_i2u2, Wup_i2u2, bup_i2u2,
           Wpost_user, bpost_user, Wpost_item, bpost_item):
    r2 = lambda b: b.reshape(1, D)
    halves = lambda x: jnp.stack([x[:, :DH], x[:, DH:]])
    e0u = ei_u2i[0].reshape(NS, SNCH, CHUNK)
    e1u = ei_u2i[1].reshape(NS, SNCH, CHUNK)
    e0i = ei_i2u[0].reshape(NS, SNCH, CHUNK)
    e1i = ei_i2u[1].reshape(NS, SNCH, CHUNK)
    f0u = ei_u2i[0].reshape(NT, NCH, CHUNK)
    f1u = ei_u2i[1].reshape(NT, NCH, CHUNK)
    f0i = ei_i2u[0].reshape(NT, NCH, CHUNK)
    f1i = ei_i2u[1].reshape(NT, NCH, CHUNK)

    seg = _SEGSUM()
    edot = _EDGEDOT()
    ps_it, pc_u2i = seg(halves(x_user), e0u, e1u)
    ps_us, pc_i2u = seg(halves(x_item), e0i, e1i)
    z_it, st_it = _DENSE_A(ps_it, pc_u2i, x_item, Wsrc_u2i1, r2(bsrc_u2i1),
                           Wdst_u2i1, r2(bdst_u2i1), Wup_u2i1, r2(bup_u2i1))
    z_us, st_us = _DENSE_A(ps_us, pc_i2u, x_user, Wsrc_i2u1, r2(bsrc_i2u1),
                           Wdst_i2u1, r2(bdst_i2u1), Wup_i2u1, r2(bup_i2u1))
    h_it = _DENSE_B1(z_it, st_it)
    h_us = _DENSE_B1(z_us, st_us)

    ps_it2, pc2 = seg(halves(h_us), e0u, e1u)
    ps_us2, pc3 = seg(halves(h_it), e0i, e1i)
    z_it2, st_it2 = _DENSE_A(ps_it2, pc2, h_it, Wsrc_u2i2, r2(bsrc_u2i2),
                             Wdst_u2i2, r2(bdst_u2i2), Wup_u2i2, r2(bup_u2i2))
    z_us2, st_us2 = _DENSE_A(ps_us2, pc3, h_us, Wsrc_i2u2, r2(bsrc_i2u2),
                             Wdst_i2u2, r2(bdst_i2u2), Wup_i2u2, r2(bup_i2u2))
    it = _DENSE_B2(z_it2, st_it2, Wpost_item, r2(bpost_item))
    u = _DENSE_B2(z_us2, st_us2, Wpost_user, r2(bpost_user))

    p0, p1 = edot(u, it, f0u, f1u, f0i, f1i)
    pr0, pr1 = _PRED(p0.reshape(_EROWS, D), p1.reshape(_EROWS, D))
    return pr0.reshape(E), pr1.reshape(E)
